# Initial kernel scaffold; baseline (speedup 1.0000x reference)
import jax
import jax.numpy as jnp
from jax.experimental import pallas as pl


def kernel(index, value, emb_table, bias):
    raise NotImplementedError("write your pallas kernel here")


# speedup vs baseline: 1.4803x; 1.4803x over previous
import functools

import jax
import jax.numpy as jnp
from jax import lax
from jax.experimental import pallas as pl
from jax.experimental.pallas import tpu as pltpu
from jax.experimental.pallas import tpu_sc as plsc

B = 16384
F = 100
V = 1000001

NC = 2
NS = 16
L = 16
NW = NC * NS

ROWS_PER_W = B // NW
CHUNK_ROWS = 128
NCHUNK = ROWS_PER_W // CHUNK_ROWS
CE = CHUNK_ROWS * F

NPIECE_FULL = V // CE
TAIL_OFF = NPIECE_FULL * CE
TAIL = V - TAIL_OFF


def kernel(index, value, emb_table, bias):
    idx_flat = index.astype(jnp.int32).reshape(-1)
    val_flat = value.reshape(-1)
    tab_flat = emb_table.reshape(-1)

    mesh = plsc.VectorSubcoreMesh(core_axis_name="c", subcore_axis_name="s")

    @functools.partial(
        pl.kernel,
        mesh=mesh,
        out_type=jax.ShapeDtypeStruct((B,), jnp.float32),
        compiler_params=pltpu.CompilerParams(needs_layout_passes=False),
        scratch_types=[
            pltpu.VMEM((CE,), jnp.int32),
            pltpu.VMEM((CE,), jnp.float32),
            pltpu.VMEM((CE,), jnp.float32),
            pltpu.VMEM((CHUNK_ROWS,), jnp.float32),
            pltpu.VMEM((L,), jnp.float32),
            pltpu.VMEM_SHARED((V,), jnp.float32),
            pltpu.SemaphoreType.DMA,
        ],
    )
    def k(idx_hbm, val_hbm, tab_hbm, bias_hbm, out_hbm,
          idx_v, val_v, gat_v, out_v, bias_v, tab_sh, sem):
        cid = lax.axis_index("c")
        sid = lax.axis_index("s")
        wid = sid * NC + cid

        for j in range((NPIECE_FULL + NS - 1) // NS):
            p = sid + j * NS

            @pl.when(p < NPIECE_FULL)
            def _():
                off = p * CE
                pltpu.sync_copy(tab_hbm.at[pl.ds(off, CE)], gat_v)
                pltpu.sync_copy(gat_v, tab_sh.at[pl.ds(off, CE)])

        @pl.when(sid == 0)
        def _():
            pltpu.sync_copy(tab_hbm.at[pl.ds(TAIL_OFF, TAIL)],
                            gat_v.at[pl.ds(0, TAIL)])
            pltpu.sync_copy(gat_v.at[pl.ds(0, TAIL)],
                            tab_sh.at[pl.ds(TAIL_OFF, TAIL)])

        pltpu.sync_copy(bias_hbm, bias_v.at[pl.ds(0, 1)])
        plsc.subcore_barrier()

        bias_s = bias_v[pl.ds(0, L)][0]
        lane_offs = lax.iota(jnp.int32, L) * F

        base_e = wid * ROWS_PER_W * F
        for kc in range(NCHUNK):
            e0 = base_e + kc * CE
            pltpu.sync_copy(idx_hbm.at[pl.ds(e0, CE)], idx_v)
            pltpu.sync_copy(val_hbm.at[pl.ds(e0, CE)], val_v)
            pltpu.async_copy(tab_sh.at[idx_v], gat_v, sem).wait()

            for g in range(CHUNK_ROWS // L):
                offs0 = lane_offs + g * (L * F)

                def body(f, carry):
                    acc, offs = carry
                    gv = plsc.load_gather(gat_v, [offs])
                    vv = plsc.load_gather(val_v, [offs])
                    return (acc + gv * vv, offs + 1)

                acc, _ = lax.fori_loop(
                    0, F, body,
                    (jnp.zeros((L,), jnp.float32), offs0), unroll=4)
                out_v[pl.ds(g * L, L)] = acc + bias_s

            pltpu.sync_copy(
                out_v,
                out_hbm.at[pl.ds(wid * ROWS_PER_W + kc * CHUNK_ROWS,
                                 CHUNK_ROWS)])

    return k(idx_flat, val_flat, tab_flat, bias)
